# Initial kernel scaffold; baseline (speedup 1.0000x reference)
import functools

import jax
import jax.numpy as jnp
from jax import lax
from jax.experimental import pallas as pl
from jax.experimental.pallas import tpu as pltpu
from jax.experimental.pallas import tpu_sc as plsc

NC = 2
NS = 16
NW = NC * NS
L = 16
C = 128
D = 128


def _build(n_edges_pad: int):
    bpw = n_edges_pad // NW
    n_chunks = bpw // C

    mesh = plsc.VectorSubcoreMesh(core_axis_name="c", subcore_axis_name="s")

    @functools.partial(
        pl.kernel,
        out_type=jax.ShapeDtypeStruct((n_edges_pad,), jnp.float32),
        mesh=mesh,
        scratch_types=[
            pltpu.VMEM((bpw,), jnp.int32),
            pltpu.VMEM((bpw,), jnp.int32),
            pltpu.VMEM((bpw,), jnp.float32),
            pltpu.VMEM((C, D), jnp.float32),
            pltpu.VMEM((C, D), jnp.float32),
            pltpu.VMEM((L, L), jnp.float32),
            pltpu.SemaphoreType.DMA,
        ],
    )
    def launch(xa, xp, ia, ip, out, ia_v, ip_v, out_v, ra_v, rp_v, m_v, sem):
        wid = lax.axis_index("s") * NC + lax.axis_index("c")
        base = wid * bpw
        pltpu.sync_copy(ia.at[pl.ds(base, bpw)], ia_v)
        pltpu.sync_copy(ip.at[pl.ds(base, bpw)], ip_v)

        row_ids = lax.iota(jnp.int32, L)

        def chunk_body(ch, _):
            off = ch * C
            pltpu.async_copy(xa.at[ia_v.at[pl.ds(off, C)]], ra_v, sem).wait()
            pltpu.async_copy(xp.at[ip_v.at[pl.ds(off, C)]], rp_v, sem).wait()

            def group_body(g, _):
                gbase = g * L
                for e in range(L):
                    r = gbase + e
                    acc = ra_v[r, pl.ds(0, L)] * rp_v[r, pl.ds(0, L)]
                    for k in range(1, D // L):
                        acc = acc + ra_v[r, pl.ds(k * L, L)] * rp_v[r, pl.ds(k * L, L)]
                    plsc.store_scatter(
                        m_v, [row_ids, jnp.full((L,), e, jnp.int32)], acc)
                s = m_v[0, :]
                for i in range(1, L):
                    s = s + m_v[i, :]
                out_v[pl.ds(off + gbase, L)] = s
                return 0

            lax.fori_loop(0, C // L, group_body, 0)
            return 0

        lax.fori_loop(0, n_chunks, chunk_body, 0)
        pltpu.sync_copy(out_v, out.at[pl.ds(base, bpw)])

    return launch


def kernel(x_author, x_paper, edge_index, supervision_edge_index):
    n = supervision_edge_index.shape[1]
    chunk_all = NW * C
    n_pad = ((n + chunk_all - 1) // chunk_all) * chunk_all
    ids = supervision_edge_index.astype(jnp.int32)
    ia = jnp.pad(ids[0], (0, n_pad - n))
    ip = jnp.pad(ids[1], (0, n_pad - n))
    scores = _build(n_pad)(x_author, x_paper, ia, ip)
    return scores[:n]


# speedup vs baseline: 1.9360x; 1.9360x over previous
import functools

import jax
import jax.numpy as jnp
from jax import lax
from jax.experimental import pallas as pl
from jax.experimental.pallas import tpu as pltpu
from jax.experimental.pallas import tpu_sc as plsc

NC = 2
NS = 16
NW = NC * NS
L = 16
C = 128
D = 128


def _shuffle(v, idx):
    dnums = lax.GatherDimensionNumbers(
        offset_dims=(), collapsed_slice_dims=(0,), start_index_map=(0,))
    return lax.gather(v, idx[:, None], dnums, slice_sizes=(1,),
                      mode=lax.GatherScatterMode.PROMISE_IN_BOUNDS)


def _build(n_edges_pad: int):
    bpw = n_edges_pad // NW
    n_chunks = bpw // C

    mesh = plsc.VectorSubcoreMesh(core_axis_name="c", subcore_axis_name="s")

    @functools.partial(
        pl.kernel,
        out_type=jax.ShapeDtypeStruct((n_edges_pad,), jnp.float32),
        mesh=mesh,
        scratch_types=[
            pltpu.VMEM((bpw,), jnp.int32),
            pltpu.VMEM((bpw,), jnp.int32),
            pltpu.VMEM((bpw,), jnp.float32),
            pltpu.VMEM((C, D), jnp.float32),
            pltpu.VMEM((C, D), jnp.float32),
            pltpu.VMEM((L, L), jnp.float32),
            pltpu.SemaphoreType.DMA,
        ],
    )
    def launch(xa, xp, ia, ip, out, ia_v, ip_v, out_v, ra_v, rp_v, m_v, sem):
        wid = lax.axis_index("s") * NC + lax.axis_index("c")
        base = wid * bpw
        pltpu.sync_copy(ia.at[pl.ds(base, bpw)], ia_v)
        pltpu.sync_copy(ip.at[pl.ds(base, bpw)], ip_v)

        row_ids = lax.iota(jnp.int32, L)

        def chunk_body(ch, _):
            off = ch * C
            pltpu.async_copy(xa.at[ia_v.at[pl.ds(off, C)]], ra_v, sem).wait()
            pltpu.async_copy(xp.at[ip_v.at[pl.ds(off, C)]], rp_v, sem).wait()

            def group_body(g, _):
                gbase = g * L
                s = jnp.zeros((L,), jnp.float32)
                for e in range(L):
                    r = gbase + e
                    acc = ra_v[r, pl.ds(0, L)] * rp_v[r, pl.ds(0, L)]
                    for k in range(1, D // L):
                        acc = acc + ra_v[r, pl.ds(k * L, L)] * rp_v[r, pl.ds(k * L, L)]
                    for sh in (8, 4, 2, 1):
                        acc = acc + _shuffle(acc, row_ids ^ sh)
                    s = jnp.where(row_ids == e, acc, s)
                out_v[pl.ds(off + gbase, L)] = s
                return 0

            lax.fori_loop(0, C // L, group_body, 0)
            return 0

        lax.fori_loop(0, n_chunks, chunk_body, 0)
        pltpu.sync_copy(out_v, out.at[pl.ds(base, bpw)])

    return launch


def kernel(x_author, x_paper, edge_index, supervision_edge_index):
    n = supervision_edge_index.shape[1]
    chunk_all = NW * C
    n_pad = ((n + chunk_all - 1) // chunk_all) * chunk_all
    ids = supervision_edge_index.astype(jnp.int32)
    ia = jnp.pad(ids[0], (0, n_pad - n))
    ip = jnp.pad(ids[1], (0, n_pad - n))
    scores = _build(n_pad)(x_author, x_paper, ia, ip)
    return scores[:n]


# speedup vs baseline: 2.4257x; 1.2529x over previous
import functools

import jax
import jax.numpy as jnp
from jax import lax
from jax.experimental import pallas as pl
from jax.experimental.pallas import tpu as pltpu
from jax.experimental.pallas import tpu_sc as plsc

NC = 2
NS = 16
NW = NC * NS
L = 16
C = 128
D = 128


def _shuffle(v, idx):
    dnums = lax.GatherDimensionNumbers(
        offset_dims=(), collapsed_slice_dims=(0,), start_index_map=(0,))
    return lax.gather(v, idx[:, None], dnums, slice_sizes=(1,),
                      mode=lax.GatherScatterMode.PROMISE_IN_BOUNDS)


def _build(n_edges_pad: int):
    bpw = n_edges_pad // NW
    n_chunks = bpw // C

    mesh = plsc.VectorSubcoreMesh(core_axis_name="c", subcore_axis_name="s")

    @functools.partial(
        pl.kernel,
        out_type=jax.ShapeDtypeStruct((n_edges_pad,), jnp.float32),
        mesh=mesh,
        scratch_types=[
            pltpu.VMEM((bpw,), jnp.int32),
            pltpu.VMEM((bpw,), jnp.int32),
            pltpu.VMEM((bpw,), jnp.float32),
            pltpu.VMEM((2, C, D), jnp.float32),
            pltpu.VMEM((2, C, D), jnp.float32),
            pltpu.SemaphoreType.DMA,
            pltpu.SemaphoreType.DMA,
        ],
    )
    def launch(xa, xp, ia, ip, out, ia_v, ip_v, out_v, ra_v, rp_v, s0, s1):
        wid = lax.axis_index("s") * NC + lax.axis_index("c")
        base = wid * bpw
        pltpu.sync_copy(ia.at[pl.ds(base, bpw)], ia_v)
        pltpu.sync_copy(ip.at[pl.ds(base, bpw)], ip_v)

        row_ids = lax.iota(jnp.int32, L)
        sems = (s0, s1)

        def descs(ch, b):
            off = ch * C
            sem = sems[b]
            return (
                pltpu.make_async_copy(
                    xa.at[ia_v.at[pl.ds(off, C)]], ra_v.at[b], sem),
                pltpu.make_async_copy(
                    xp.at[ip_v.at[pl.ds(off, C)]], rp_v.at[b], sem),
            )

        def issue(ch, b):
            for d in descs(ch, b):
                d.start()

        def wait(ch, b):
            for d in descs(ch, b):
                d.wait()

        def compute(ch, b):
            ra = ra_v.at[b]
            rp = rp_v.at[b]
            off = ch * C

            def group_body(g, _):
                gbase = g * L
                s = jnp.zeros((L,), jnp.float32)
                for e in range(L):
                    r = gbase + e
                    acc = ra[r, pl.ds(0, L)] * rp[r, pl.ds(0, L)]
                    for k in range(1, D // L):
                        acc = acc + ra[r, pl.ds(k * L, L)] * rp[r, pl.ds(k * L, L)]
                    for sh in (8, 4, 2, 1):
                        acc = acc + _shuffle(acc, row_ids ^ sh)
                    s = jnp.where(row_ids == e, acc, s)
                out_v[pl.ds(off + gbase, L)] = s
                return 0

            lax.fori_loop(0, C // L, group_body, 0)

        issue(0, 0)

        def pair_body(i, _):
            ch = 2 * i
            wait(ch, 0)
            issue(ch + 1, 1)
            compute(ch, 0)
            wait(ch + 1, 1)

            @pl.when(ch + 2 < n_chunks)
            def _():
                issue(ch + 2, 0)

            compute(ch + 1, 1)
            return 0

        lax.fori_loop(0, n_chunks // 2, pair_body, 0)
        pltpu.sync_copy(out_v, out.at[pl.ds(base, bpw)])

    return launch


def kernel(x_author, x_paper, edge_index, supervision_edge_index):
    n = supervision_edge_index.shape[1]
    chunk_all = NW * C * 2
    n_pad = ((n + chunk_all - 1) // chunk_all) * chunk_all
    ids = supervision_edge_index.astype(jnp.int32)
    ia = jnp.pad(ids[0], (0, n_pad - n))
    ip = jnp.pad(ids[1], (0, n_pad - n))
    scores = _build(n_pad)(x_author, x_paper, ia, ip)
    return scores[:n]
